# Initial kernel scaffold; baseline (speedup 1.0000x reference)
import jax
import jax.numpy as jnp
from jax.experimental import pallas as pl


def kernel(x, edge_index, edge_attr, batch, atom_tables, bond_tables, W, b, root, bn_gamma, bn_beta, W_out, b_out):
    raise NotImplementedError("write your pallas kernel here")


# speedup vs baseline: 4.4672x; 4.4672x over previous
import functools
import jax
import jax.numpy as jnp
from jax import lax
from jax.experimental import pallas as pl
from jax.experimental.pallas import tpu as pltpu
from jax.experimental.pallas import tpu_sc as plsc

N = 10000
E = 320000
D = 128
G = 64

NC = 2
NS = 16
NW = NC * NS

NP = 10240
ROWS_PER_TILE = NP // NS

WIN = 80
E_PER_W = E // NW
NWIN = E_PER_W // WIN
E_PER_T = E // NS
NWIN_DEG = E_PER_T // WIN

@functools.cache
def _mesh():
    return plsc.VectorSubcoreMesh(
        core_axis_name="c", subcore_axis_name="s",
        num_cores=NC, num_subcores=NS)


LUTSZ = E + 8


def _sc_kernel(**kw):
    def deco(body):
        @functools.cache
        def build():
            return pl.kernel(
                body, mesh=_mesh(),
                compiler_params=pltpu.CompilerParams(
                    needs_layout_passes=False,
                    use_tc_tiling_on_sc=False),
                **kw)

        def call(*args):
            return build()(*args)
        return call
    return deco


@_sc_kernel(
    out_type=(
        jax.ShapeDtypeStruct((NW, NWIN, WIN), jnp.float32),
        jax.ShapeDtypeStruct((NP,), jnp.float32),
    ),
    scratch_types=[
        pltpu.VMEM((NWIN_DEG, WIN), jnp.int32),
        pltpu.VMEM((WIN,), jnp.float32),
        pltpu.VMEM((ROWS_PER_TILE,), jnp.float32),
        pltpu.VMEM((ROWS_PER_TILE,), jnp.int32),
        pltpu.VMEM((ROWS_PER_TILE,), jnp.float32),
        pltpu.VMEM((NWIN, WIN), jnp.int32),
        pltpu.VMEM((NWIN, WIN), jnp.int32),
        pltpu.VMEM((NWIN, WIN), jnp.float32),
        pltpu.VMEM((NP,), jnp.float32),
        pltpu.VMEM_SHARED((NP,), jnp.float32),
        pltpu.VMEM_SHARED((NP,), jnp.float32),
        pltpu.SemaphoreType.DMA,
    ],
)
def _prep_kernel(rowd3_hbm, row3_hbm, col3_hbm, lut_hbm, norm_out, deg_out,
                 rowwin_v, ones_v, deg_v, degi_v, dis_v, rown_v, coln_v,
                 normn_v, disfull_v, deg_sh, dis_sh, sem):
    c = lax.axis_index("c")
    s = lax.axis_index("s")
    wid = s * NC + c

    def _init16(i, _):
        ones_v[pl.ds(i * 16, 16)] = jnp.ones((16,), jnp.float32)
        return 0
    lax.fori_loop(0, WIN // 16, _init16, 0)

    def _zero16(i, _):
        deg_v[pl.ds(i * 16, 16)] = jnp.zeros((16,), jnp.float32)
        return 0
    lax.fori_loop(0, ROWS_PER_TILE // 16, _zero16, 0)
    pltpu.sync_copy(deg_v, deg_sh.at[pl.ds(s * ROWS_PER_TILE, ROWS_PER_TILE)])
    plsc.subcore_barrier()

    pltpu.sync_copy(rowd3_hbm.at[s], rowwin_v)

    def _deg_win(w, _):
        pltpu.sync_copy(ones_v, deg_sh.at[rowwin_v.at[w]], add=True)
        return 0
    lax.fori_loop(0, NWIN_DEG, _deg_win, 0)
    plsc.subcore_barrier()

    pltpu.sync_copy(deg_sh.at[pl.ds(s * ROWS_PER_TILE, ROWS_PER_TILE)], deg_v)

    def _dis16(i, _):
        x = deg_v[pl.ds(i * 16, 16)] + 1.0
        deg_v[pl.ds(i * 16, 16)] = x
        degi_v[pl.ds(i * 16, 16)] = x.astype(jnp.int32)
        return 0
    lax.fori_loop(0, ROWS_PER_TILE // 16, _dis16, 0)
    for k in range(ROWS_PER_TILE // 128):
        pltpu.async_copy(
            lut_hbm.at[degi_v.at[pl.ds(k * 128, 128)]],
            dis_v.at[pl.ds(k * 128, 128)], sem).wait()
    pltpu.sync_copy(dis_v, dis_sh.at[pl.ds(s * ROWS_PER_TILE, ROWS_PER_TILE)])

    @pl.when(c == 0)
    def _():
        pltpu.sync_copy(deg_v, deg_out.at[pl.ds(s * ROWS_PER_TILE, ROWS_PER_TILE)])
    plsc.subcore_barrier()

    pltpu.sync_copy(dis_sh, disfull_v)
    pltpu.sync_copy(row3_hbm.at[wid], rown_v)
    pltpu.sync_copy(col3_hbm.at[wid], coln_v)

    def _normw(w, _):
        for g in range(WIN // 16):
            r16 = rown_v[w, pl.ds(g * 16, 16)]
            c16 = coln_v[w, pl.ds(g * 16, 16)]
            dr = plsc.load_gather(disfull_v, [r16])
            dc = plsc.load_gather(disfull_v, [c16])
            normn_v[w, pl.ds(g * 16, 16)] = dr * dc
        return 0
    lax.fori_loop(0, NWIN, _normw, 0)
    pltpu.sync_copy(normn_v, norm_out.at[wid])


DH = D // 2


@_sc_kernel(
    out_type=jax.ShapeDtypeStruct((NC, 2, NP, DH), jnp.float32),
    scratch_types=[
        pltpu.VMEM((NWIN, WIN), jnp.int32),
        pltpu.VMEM((NWIN, WIN), jnp.int32),
        pltpu.VMEM((NWIN, WIN), jnp.int32),
        pltpu.VMEM((NWIN, WIN), jnp.float32),
        pltpu.VMEM((WIN, DH), jnp.float32),
        pltpu.VMEM((WIN, DH), jnp.float32),
        pltpu.VMEM((WIN, DH), jnp.float32),
        pltpu.VMEM_SHARED((NP, DH), jnp.float32),
        pltpu.SemaphoreType.DMA,
        pltpu.SemaphoreType.DMA,
    ],
)
def _msg_kernel(hwlo_hbm, hwhi_hbm, tlo_hbm, thi_hbm, row3d_hbm, col3d_hbm,
                code3d_hbm, norm3d_hbm, out_hbm, roww_v, colw_v, codew_v,
                normw_v, hbuf, ebuf, msgbuf, agg_sh, sem1, sem2):
    c = lax.axis_index("c")
    s = lax.axis_index("s")
    wid = s * NC + c

    pltpu.sync_copy(row3d_hbm.at[wid], roww_v)
    pltpu.sync_copy(col3d_hbm.at[wid], colw_v)
    pltpu.sync_copy(code3d_hbm.at[wid], codew_v)
    pltpu.sync_copy(norm3d_hbm.at[wid], normw_v)

    for dh, hw_hbm, t_hbm in ((0, hwlo_hbm, tlo_hbm), (1, hwhi_hbm, thi_hbm)):
        def _z(i, _):
            for j in range(DH // 16):
                msgbuf[i, pl.ds(j * 16, 16)] = jnp.zeros((16,), jnp.float32)
            return 0
        lax.fori_loop(0, WIN, _z, 0)
        for k in range(ROWS_PER_TILE // WIN):
            pltpu.sync_copy(
                msgbuf, agg_sh.at[pl.ds(s * ROWS_PER_TILE + k * WIN, WIN)])
        plsc.subcore_barrier()

        def _win(w, _):
            cp1 = pltpu.async_copy(hw_hbm.at[roww_v.at[w]], hbuf, sem1)
            cp2 = pltpu.async_copy(t_hbm.at[codew_v.at[w]], ebuf, sem2)
            cp1.wait()
            cp2.wait()

            def _edge(e, _):
                nv = plsc.load_gather(
                    normw_v, [jnp.full((16,), w, jnp.int32),
                              jnp.full((16,), e, jnp.int32)])
                for j in range(DH // 16):
                    v = hbuf[e, pl.ds(j * 16, 16)] + ebuf[e, pl.ds(j * 16, 16)]
                    msgbuf[e, pl.ds(j * 16, 16)] = jnp.maximum(v, 0.0) * nv
                return 0
            lax.fori_loop(0, WIN, _edge, 0)

            pltpu.sync_copy(msgbuf, agg_sh.at[colw_v.at[w]], add=True)
            return 0
        lax.fori_loop(0, NWIN, _win, 0)
        plsc.subcore_barrier()

        pltpu.sync_copy(
            agg_sh.at[pl.ds(s * ROWS_PER_TILE, ROWS_PER_TILE)],
            out_hbm.at[c, dh].at[pl.ds(s * ROWS_PER_TILE, ROWS_PER_TILE)])


BLK = 2000


def _atom_mm_body(x_ref, tab_ref, wt_ref, b_ref, root_ref, deg_ref,
                  hwlo_ref, hwhi_ref, s_ref):
    x = x_ref[...]
    h = jnp.zeros((BLK, D), jnp.float32)
    iota = lax.broadcasted_iota(jnp.int32, (1, 64), 1)
    for k in range(9):
        oh = (x[:, k:k + 1] == iota).astype(jnp.float32)
        h = h + jnp.dot(oh, tab_ref[pl.ds(k * 64, 64), :],
                        preferred_element_type=jnp.float32)
    hw = jnp.dot(h, wt_ref[...], preferred_element_type=jnp.float32) + b_ref[...]
    hwlo_ref[...] = hw[:, :DH]
    hwhi_ref[...] = hw[:, DH:]
    s_ref[...] = jnp.maximum(hw + root_ref[...], 0.0) / deg_ref[...]


def _ep_mm_body(a00_ref, a01_ref, a10_ref, a11_ref, sp_ref, scale_ref,
                shift_ref, wt_ref, b_ref, root_ref, deg_ref,
                hwlo_ref, hwhi_ref, s_ref):
    a = jnp.concatenate(
        [a00_ref[...] + a10_ref[...], a01_ref[...] + a11_ref[...]], axis=1)
    h = a + sp_ref[...]
    h = jnp.maximum(h * scale_ref[...] + shift_ref[...], 0.0)
    hw = jnp.dot(h, wt_ref[...], preferred_element_type=jnp.float32) + b_ref[...]
    hwlo_ref[...] = hw[:, :DH]
    hwhi_ref[...] = hw[:, DH:]
    s_ref[...] = jnp.maximum(hw + root_ref[...], 0.0) / deg_ref[...]


def _pool_body(batch_ref, a00_ref, a01_ref, a10_ref, a11_ref, sp_ref, wt_ref,
               b_ref, out_ref, accp, accc):
    step = pl.program_id(0)

    @pl.when(step == 0)
    def _():
        accp[...] = jnp.zeros((G, D), jnp.float32)
        accc[...] = jnp.zeros((8, G), jnp.float32)

    a = jnp.concatenate(
        [a00_ref[...] + a10_ref[...], a01_ref[...] + a11_ref[...]], axis=1)
    h3 = a + sp_ref[...]
    iota = lax.broadcasted_iota(jnp.int32, (1, G), 1)
    oh = (batch_ref[...] == iota).astype(jnp.float32)
    accp[...] += jnp.dot(oh.T, h3, preferred_element_type=jnp.float32)
    accc[...] += jnp.sum(oh.reshape(BLK // 8, 8, G), axis=0)

    @pl.when(step == pl.num_programs(0) - 1)
    def _():
        cnt = jnp.sum(accc[...], axis=0)[:, None]
        pooled = accp[...] / jnp.maximum(cnt, 1.0)
        out_ref[...] = jnp.dot(pooled, wt_ref[...],
                               preferred_element_type=jnp.float32) + b_ref[...]


def _row_spec(blk, d=D):
    return pl.BlockSpec((blk, d), lambda i: (i, 0))


def _full_spec(shape):
    return pl.BlockSpec(shape, lambda i: tuple(0 for _ in shape))


def _atom_mm(x, atom_flat, wt, bias, rootv, deg):
    grid = (N // BLK,)
    return pl.pallas_call(
        _atom_mm_body,
        grid=grid,
        in_specs=[
            pl.BlockSpec((BLK, 16), lambda i: (i, 0)),
            _full_spec((576, D)),
            _full_spec((D, D)),
            _full_spec((1, D)),
            _full_spec((1, D)),
            pl.BlockSpec((BLK, 1), lambda i: (i, 0)),
        ],
        out_specs=[_row_spec(BLK, DH), _row_spec(BLK, DH), _row_spec(BLK)],
        out_shape=[jax.ShapeDtypeStruct((N, DH), jnp.float32),
                   jax.ShapeDtypeStruct((N, DH), jnp.float32),
                   jax.ShapeDtypeStruct((N, D), jnp.float32)],
    )(x, atom_flat, wt, bias, rootv, deg)


def _ep_mm(a00, a01, a10, a11, sp, scale, shift, wt, bias, rootv, deg):
    grid = (N // BLK,)
    return pl.pallas_call(
        _ep_mm_body,
        grid=grid,
        in_specs=[
            _row_spec(BLK, DH), _row_spec(BLK, DH),
            _row_spec(BLK, DH), _row_spec(BLK, DH),
            _row_spec(BLK),
            _full_spec((1, D)), _full_spec((1, D)),
            _full_spec((D, D)), _full_spec((1, D)), _full_spec((1, D)),
            pl.BlockSpec((BLK, 1), lambda i: (i, 0)),
        ],
        out_specs=[_row_spec(BLK, DH), _row_spec(BLK, DH), _row_spec(BLK)],
        out_shape=[jax.ShapeDtypeStruct((N, DH), jnp.float32),
                   jax.ShapeDtypeStruct((N, DH), jnp.float32),
                   jax.ShapeDtypeStruct((N, D), jnp.float32)],
    )(a00, a01, a10, a11, sp, scale, shift, wt, bias, rootv, deg)


def _pool(batch2, a00, a01, a10, a11, sp, wt, bias):
    grid = (N // BLK,)
    return pl.pallas_call(
        _pool_body,
        grid=grid,
        in_specs=[
            pl.BlockSpec((BLK, 1), lambda i: (i, 0)),
            _row_spec(BLK, DH), _row_spec(BLK, DH),
            _row_spec(BLK, DH), _row_spec(BLK, DH),
            _row_spec(BLK),
            _full_spec((D, D)), _full_spec((1, D)),
        ],
        out_specs=pl.BlockSpec((G, D), lambda i: (0, 0)),
        out_shape=jax.ShapeDtypeStruct((G, D), jnp.float32),
        scratch_shapes=[
            pltpu.VMEM((G, D), jnp.float32),
            pltpu.VMEM((8, G), jnp.float32),
        ],
    )(batch2, a00, a01, a10, a11, sp, wt, bias)


def kernel(x, edge_index, edge_attr, batch, atom_tables, bond_tables, W, b,
           root, bn_gamma, bn_beta, W_out, b_out):
    row = edge_index[0].astype(jnp.int32)
    col = edge_index[1].astype(jnp.int32)
    ea = edge_attr.astype(jnp.int32)
    code = ea[:, 0] * 64 + ea[:, 1] * 8 + ea[:, 2]

    T = (bond_tables[:, 0][:, :, None, None, :]
         + bond_tables[:, 1][:, None, :, None, :]
         + bond_tables[:, 2][:, None, None, :, :]).reshape(3, 512, D)
    t_lo = T[:, :, :DH]
    t_hi = T[:, :, DH:]
    atom_flat = atom_tables.reshape(576, D)

    rowd3 = lax.optimization_barrier(row.reshape(NS, NWIN_DEG, WIN))
    row3 = row.reshape(NW, NWIN, WIN)
    col3 = col.reshape(NW, NWIN, WIN)
    code3 = code.reshape(NW, NWIN, WIN)

    lut = jnp.where(jnp.arange(LUTSZ) > 0,
                    jax.lax.rsqrt(jnp.arange(LUTSZ, dtype=jnp.float32)), 0.0)
    norm3, degp = _prep_kernel(rowd3, row3, col3, lut)
    deg = degp[:N, None]

    x16 = jnp.zeros((N, 16), jnp.int32).at[:, :9].set(x.astype(jnp.int32))

    bias = b.astype(jnp.float32)
    scale = (bn_gamma / jnp.sqrt(1.0 + 1e-5)).astype(jnp.float32)

    hwlo, hwhi, sterm = _atom_mm(x16, atom_flat, W[0].T, bias[0][None],
                                 root[0][None], deg)

    wt_p = jnp.concatenate(
        [jnp.swapaxes(W, 1, 2), jnp.zeros((1, D, D), jnp.float32)], 0)
    bias_p = jnp.concatenate([bias, jnp.zeros((1, D), jnp.float32)], 0)
    root_p = jnp.concatenate(
        [root.astype(jnp.float32), jnp.zeros((1, D), jnp.float32)], 0)
    scale_p = jnp.concatenate([scale, jnp.ones((1, D), jnp.float32)], 0)
    beta_p = jnp.concatenate(
        [bn_beta.astype(jnp.float32), jnp.zeros((1, D), jnp.float32)], 0)

    zero_nh = jnp.zeros((N, DH), jnp.float32)

    def body(i, carry):
        hwlo, hwhi, sterm = carry[0], carry[1], carry[2]
        agg = _msg_kernel(hwlo, hwhi, t_lo[i], t_hi[i],
                          row3, col3, code3, norm3)
        a00 = agg[0, 0, :N]
        a01 = agg[0, 1, :N]
        a10 = agg[1, 0, :N]
        a11 = agg[1, 1, :N]
        hwlo2, hwhi2, sterm2 = _ep_mm(
            a00, a01, a10, a11, sterm, scale_p[i][None], beta_p[i][None],
            wt_p[i + 1], bias_p[i + 1][None], root_p[i + 1][None], deg)
        return (hwlo2, hwhi2, sterm2, a00, a01, a10, a11, sterm)

    ub = lax.optimization_barrier(jnp.int32(3))
    res = lax.fori_loop(
        0, ub, body,
        (hwlo, hwhi, sterm, zero_nh, zero_nh, zero_nh, zero_nh,
         jnp.zeros((N, D), jnp.float32)))
    _, _, _, a00, a01, a10, a11, sterm_last = res
    return _pool(batch.astype(jnp.int32)[:, None], a00, a01, a10, a11,
                 sterm_last, W_out.T, b_out[None])
